# Initial kernel scaffold; baseline (speedup 1.0000x reference)
import jax
import jax.numpy as jnp
from jax.experimental import pallas as pl


def kernel(h, edge_index):
    raise NotImplementedError("write your pallas kernel here")


# speedup vs baseline: 1.1954x; 1.1954x over previous
import functools

import jax
import jax.numpy as jnp
from jax import lax
from jax.experimental import pallas as pl
from jax.experimental.pallas import tpu as pltpu
from jax.experimental.pallas import tpu_sc as plsc

D_FEAT = 128
NUM_WORKERS = 32
CHUNK = 400


@functools.partial(jax.jit, static_argnames=("n_edges",))
def _dot_predict_sc(h, src, dst, n_edges):
    per_w = n_edges // NUM_WORKERS
    n_chunks = per_w // CHUNK
    mesh = plsc.VectorSubcoreMesh(core_axis_name="c", subcore_axis_name="s")

    @functools.partial(
        pl.kernel,
        mesh=mesh,
        compiler_params=pltpu.CompilerParams(needs_layout_passes=False),
        out_type=jax.ShapeDtypeStruct((n_edges,), jnp.float32),
        scratch_types=[
            pltpu.VMEM((CHUNK,), jnp.int32),
            pltpu.VMEM((CHUNK,), jnp.int32),
            pltpu.VMEM((CHUNK, D_FEAT), jnp.float32),
            pltpu.VMEM((CHUNK, D_FEAT), jnp.float32),
            pltpu.VMEM((CHUNK,), jnp.float32),
            pltpu.SemaphoreType.DMA,
            pltpu.SemaphoreType.DMA,
        ],
    )
    def sc_kernel(h_hbm, src_hbm, dst_hbm, out_hbm,
                  sidx, didx, srows, drows, outv, sem_s, sem_d):
        wid = lax.axis_index("s") * 2 + lax.axis_index("c")
        wbase = wid * per_w

        def chunk_body(ci, carry):
            base = wbase + ci * CHUNK
            pltpu.sync_copy(src_hbm.at[pl.ds(base, CHUNK)], sidx)
            pltpu.sync_copy(dst_hbm.at[pl.ds(base, CHUNK)], didx)
            cp_s = pltpu.async_copy(h_hbm.at[sidx], srows, sem_s)
            cp_d = pltpu.async_copy(h_hbm.at[didx], drows, sem_d)
            cp_s.wait()
            cp_d.wait()

            def block_body(b, carry2):
                rows = b * 16 + lax.iota(jnp.int32, 16)

                def k_body(kk, acc):
                    cols = jnp.full((16,), kk, jnp.int32)
                    a = plsc.load_gather(srows, [rows, cols])
                    bb = plsc.load_gather(drows, [rows, cols])
                    return acc + a * bb

                acc = lax.fori_loop(0, D_FEAT, k_body,
                                    jnp.zeros((16,), jnp.float32))
                outv[pl.ds(b * 16, 16)] = acc
                return carry2

            lax.fori_loop(0, CHUNK // 16, block_body, 0)
            pltpu.sync_copy(outv, out_hbm.at[pl.ds(base, CHUNK)])
            return carry

        lax.fori_loop(0, n_chunks, chunk_body, 0)

    return sc_kernel(h, src, dst)


def kernel(h, edge_index):
    src = edge_index[0].astype(jnp.int32)
    dst = edge_index[1].astype(jnp.int32)
    return _dot_predict_sc(h, src, dst, src.shape[0])


# speedup vs baseline: 1.3424x; 1.1230x over previous
import functools

import jax
import jax.numpy as jnp
from jax import lax
from jax.experimental import pallas as pl
from jax.experimental.pallas import tpu as pltpu
from jax.experimental.pallas import tpu_sc as plsc

D_FEAT = 128
NUM_WORKERS = 32
CHUNK = 80


@functools.partial(jax.jit, static_argnames=("n_edges",))
def _dot_predict_sc(h, src, dst, n_edges):
    per_w = n_edges // NUM_WORKERS
    n_chunks = per_w // CHUNK
    mesh = plsc.VectorSubcoreMesh(core_axis_name="c", subcore_axis_name="s")

    @functools.partial(
        pl.kernel,
        mesh=mesh,
        compiler_params=pltpu.CompilerParams(needs_layout_passes=False),
        out_type=jax.ShapeDtypeStruct((n_edges,), jnp.float32),
        scratch_types=[
            pltpu.VMEM((per_w,), jnp.int32),
            pltpu.VMEM((per_w,), jnp.int32),
            pltpu.VMEM((CHUNK, D_FEAT), jnp.float32),
            pltpu.VMEM((CHUNK, D_FEAT), jnp.float32),
            pltpu.VMEM((CHUNK, D_FEAT), jnp.float32),
            pltpu.VMEM((CHUNK, D_FEAT), jnp.float32),
            pltpu.VMEM((per_w,), jnp.float32),
            pltpu.SemaphoreType.DMA,
            pltpu.SemaphoreType.DMA,
            pltpu.SemaphoreType.DMA,
            pltpu.SemaphoreType.DMA,
        ],
    )
    def sc_kernel(h_hbm, src_hbm, dst_hbm, out_hbm,
                  sidx, didx, srows0, drows0, srows1, drows1, outv,
                  sem_s0, sem_d0, sem_s1, sem_d1):
        wid = lax.axis_index("s") * 2 + lax.axis_index("c")
        wbase = wid * per_w
        srows = (srows0, srows1)
        drows = (drows0, drows1)
        sem_s = (sem_s0, sem_s1)
        sem_d = (sem_d0, sem_d1)

        pltpu.sync_copy(src_hbm.at[pl.ds(wbase, per_w)], sidx)
        pltpu.sync_copy(dst_hbm.at[pl.ds(wbase, per_w)], didx)

        def issue(ci, buf):
            pltpu.async_copy(h_hbm.at[sidx.at[pl.ds(ci * CHUNK, CHUNK)]],
                             srows[buf], sem_s[buf])
            pltpu.async_copy(h_hbm.at[didx.at[pl.ds(ci * CHUNK, CHUNK)]],
                             drows[buf], sem_d[buf])

        def wait(buf):
            pltpu.make_async_copy(h_hbm.at[sidx.at[pl.ds(0, CHUNK)]],
                                  srows[buf], sem_s[buf]).wait()
            pltpu.make_async_copy(h_hbm.at[didx.at[pl.ds(0, CHUNK)]],
                                  drows[buf], sem_d[buf]).wait()

        def compute(ci, buf):
            def block_body(b, carry):
                rows = b * 16 + lax.iota(jnp.int32, 16)

                def k_body(kk, acc):
                    cols = jnp.full((16,), kk, jnp.int32)
                    a = plsc.load_gather(srows[buf], [rows, cols])
                    bb = plsc.load_gather(drows[buf], [rows, cols])
                    return acc + a * bb

                acc = lax.fori_loop(0, D_FEAT, k_body,
                                    jnp.zeros((16,), jnp.float32),
                                    unroll=32)
                outv[pl.ds(ci * CHUNK + b * 16, 16)] = acc
                return carry

            lax.fori_loop(0, CHUNK // 16, block_body, 0)

        issue(0, 0)

        def pair_body(g, carry):
            for b in (0, 1):
                ci = g * 2 + b
                issue(ci + 1, 1 - b)
                wait(b)
                compute(ci, b)
            return carry

        lax.fori_loop(0, (n_chunks - 1) // 2, pair_body, 0)
        wait(0)
        compute(n_chunks - 1, 0)

        pltpu.sync_copy(outv, out_hbm.at[pl.ds(wbase, per_w)])

    return sc_kernel(h, src, dst)


def kernel(h, edge_index):
    src = edge_index[0].astype(jnp.int32)
    dst = edge_index[1].astype(jnp.int32)
    return _dot_predict_sc(h, src, dst, src.shape[0])


# speedup vs baseline: 9.2741x; 6.9085x over previous
import functools

import jax
import jax.numpy as jnp
from jax import lax
from jax.experimental import pallas as pl
from jax.experimental.pallas import tpu as pltpu
from jax.experimental.pallas import tpu_sc as plsc

D_FEAT = 128
NUM_WORKERS = 32
CHUNK = 80


@functools.partial(jax.jit, static_argnames=("n_edges",))
def _dot_predict_sc(h, src, dst, n_edges):
    per_w = n_edges // NUM_WORKERS
    n_chunks = per_w // CHUNK
    mesh = plsc.VectorSubcoreMesh(core_axis_name="c", subcore_axis_name="s")

    @functools.partial(
        pl.kernel,
        mesh=mesh,
        compiler_params=pltpu.CompilerParams(needs_layout_passes=False),
        out_type=jax.ShapeDtypeStruct((n_edges,), jnp.float32),
        scratch_types=[
            pltpu.VMEM((per_w,), jnp.int32),
            pltpu.VMEM((per_w,), jnp.int32),
            pltpu.VMEM((CHUNK, D_FEAT), jnp.float32),
            pltpu.VMEM((CHUNK, D_FEAT), jnp.float32),
            pltpu.VMEM((CHUNK, D_FEAT), jnp.float32),
            pltpu.VMEM((CHUNK, D_FEAT), jnp.float32),
            pltpu.VMEM((per_w,), jnp.float32),
            pltpu.SemaphoreType.DMA,
            pltpu.SemaphoreType.DMA,
            pltpu.SemaphoreType.DMA,
            pltpu.SemaphoreType.DMA,
        ],
    )
    def sc_kernel(h_hbm, src_hbm, dst_hbm, out_hbm,
                  sidx, didx, srows0, drows0, srows1, drows1, outv,
                  sem_s0, sem_d0, sem_s1, sem_d1):
        wid = lax.axis_index("s") * 2 + lax.axis_index("c")
        wbase = wid * per_w
        srows = (srows0, srows1)
        drows = (drows0, drows1)
        sem_s = (sem_s0, sem_s1)
        sem_d = (sem_d0, sem_d1)

        pltpu.sync_copy(src_hbm.at[pl.ds(wbase, per_w)], sidx)
        pltpu.sync_copy(dst_hbm.at[pl.ds(wbase, per_w)], didx)

        def issue(ci, buf):
            pltpu.async_copy(h_hbm.at[sidx.at[pl.ds(ci * CHUNK, CHUNK)]],
                             srows[buf], sem_s[buf])
            pltpu.async_copy(h_hbm.at[didx.at[pl.ds(ci * CHUNK, CHUNK)]],
                             drows[buf], sem_d[buf])

        def wait(buf):
            pltpu.make_async_copy(h_hbm.at[sidx.at[pl.ds(0, CHUNK)]],
                                  srows[buf], sem_s[buf]).wait()
            pltpu.make_async_copy(h_hbm.at[didx.at[pl.ds(0, CHUNK)]],
                                  drows[buf], sem_d[buf]).wait()

        def compute(ci, buf):
            lane = lax.iota(jnp.int32, 16)

            def block_body(b, carry):
                rows = b * 16 + lane

                def k_body(kk, acc):
                    cols = (lane + kk) & (D_FEAT - 1)
                    a = plsc.load_gather(srows[buf], [rows, cols])
                    bb = plsc.load_gather(drows[buf], [rows, cols])
                    return acc + a * bb

                acc = lax.fori_loop(0, D_FEAT, k_body,
                                    jnp.zeros((16,), jnp.float32),
                                    unroll=32)
                outv[pl.ds(ci * CHUNK + b * 16, 16)] = acc
                return carry

            lax.fori_loop(0, CHUNK // 16, block_body, 0)

        issue(0, 0)

        def pair_body(g, carry):
            for b in (0, 1):
                ci = g * 2 + b
                issue(ci + 1, 1 - b)
                wait(b)
                compute(ci, b)
            return carry

        lax.fori_loop(0, (n_chunks - 1) // 2, pair_body, 0)
        wait(0)
        compute(n_chunks - 1, 0)

        pltpu.sync_copy(outv, out_hbm.at[pl.ds(wbase, per_w)])

    return sc_kernel(h, src, dst)


def kernel(h, edge_index):
    src = edge_index[0].astype(jnp.int32)
    dst = edge_index[1].astype(jnp.int32)
    return _dot_predict_sc(h, src, dst, src.shape[0])


# speedup vs baseline: 10.8988x; 1.1752x over previous
import functools

import jax
import jax.numpy as jnp
from jax import lax
from jax.experimental import pallas as pl
from jax.experimental.pallas import tpu as pltpu
from jax.experimental.pallas import tpu_sc as plsc

D_FEAT = 128
NUM_WORKERS = 32
CHUNK = 80


@functools.partial(jax.jit, static_argnames=("n_edges",))
def _dot_predict_sc(h, src, dst, n_edges):
    per_w = n_edges // NUM_WORKERS
    n_chunks = per_w // CHUNK
    mesh = plsc.VectorSubcoreMesh(core_axis_name="c", subcore_axis_name="s")

    @functools.partial(
        pl.kernel,
        mesh=mesh,
        compiler_params=pltpu.CompilerParams(needs_layout_passes=False),
        out_type=jax.ShapeDtypeStruct((n_edges,), jnp.float32),
        scratch_types=[
            pltpu.VMEM((per_w,), jnp.int32),
            pltpu.VMEM((per_w,), jnp.int32),
            pltpu.VMEM((CHUNK, D_FEAT), jnp.float32),
            pltpu.VMEM((CHUNK, D_FEAT), jnp.float32),
            pltpu.VMEM((CHUNK, D_FEAT), jnp.float32),
            pltpu.VMEM((CHUNK, D_FEAT), jnp.float32),
            pltpu.VMEM((CHUNK, D_FEAT), jnp.float32),
            pltpu.VMEM((CHUNK, D_FEAT), jnp.float32),
            pltpu.VMEM((CHUNK, D_FEAT), jnp.float32),
            pltpu.VMEM((CHUNK, D_FEAT), jnp.float32),
            pltpu.VMEM((per_w,), jnp.float32),
            pltpu.SemaphoreType.DMA,
            pltpu.SemaphoreType.DMA,
            pltpu.SemaphoreType.DMA,
            pltpu.SemaphoreType.DMA,
            pltpu.SemaphoreType.DMA,
            pltpu.SemaphoreType.DMA,
            pltpu.SemaphoreType.DMA,
            pltpu.SemaphoreType.DMA,
        ],
    )
    def sc_kernel(h_hbm, src_hbm, dst_hbm, out_hbm,
                  sidx, didx, srows0, drows0, srows1, drows1,
                  srows2, drows2, srows3, drows3, outv,
                  sem_s0, sem_d0, sem_s1, sem_d1,
                  sem_s2, sem_d2, sem_s3, sem_d3):
        wid = lax.axis_index("s") * 2 + lax.axis_index("c")
        wbase = wid * per_w
        srows = (srows0, srows1, srows2, srows3)
        drows = (drows0, drows1, drows2, drows3)
        sem_s = (sem_s0, sem_s1, sem_s2, sem_s3)
        sem_d = (sem_d0, sem_d1, sem_d2, sem_d3)

        pltpu.sync_copy(src_hbm.at[pl.ds(wbase, per_w)], sidx)
        pltpu.sync_copy(dst_hbm.at[pl.ds(wbase, per_w)], didx)

        def issue(ci, buf):
            pltpu.async_copy(h_hbm.at[sidx.at[pl.ds(ci * CHUNK, CHUNK)]],
                             srows[buf], sem_s[buf])
            pltpu.async_copy(h_hbm.at[didx.at[pl.ds(ci * CHUNK, CHUNK)]],
                             drows[buf], sem_d[buf])

        def wait(buf):
            pltpu.make_async_copy(h_hbm.at[sidx.at[pl.ds(0, CHUNK)]],
                                  srows[buf], sem_s[buf]).wait()
            pltpu.make_async_copy(h_hbm.at[didx.at[pl.ds(0, CHUNK)]],
                                  drows[buf], sem_d[buf]).wait()

        def compute(ci, buf):
            lane = lax.iota(jnp.int32, 16)

            def block_body(b, carry):
                rows = b * 16 + lane

                def k_body(kk, acc):
                    cols = (lane + kk) & (D_FEAT - 1)
                    a = plsc.load_gather(srows[buf], [rows, cols])
                    bb = plsc.load_gather(drows[buf], [rows, cols])
                    return acc + a * bb

                acc = lax.fori_loop(0, D_FEAT, k_body,
                                    jnp.zeros((16,), jnp.float32),
                                    unroll=32)
                outv[pl.ds(ci * CHUNK + b * 16, 16)] = acc
                return carry

            lax.fori_loop(0, CHUNK // 16, block_body, 0)

        issue(0, 0)
        issue(1, 1)
        issue(2, 2)

        def quad_body(g, carry):
            for b in (0, 1, 2, 3):
                ci = g * 4 + b

                @pl.when(ci + 3 < n_chunks)
                def _issue_ahead():
                    issue(ci + 3, (b + 3) % 4)

                wait(b)
                compute(ci, b)
            return carry

        lax.fori_loop(0, (n_chunks - 1) // 4, quad_body, 0)
        wait(0)
        compute(n_chunks - 1, 0)

        pltpu.sync_copy(outv, out_hbm.at[pl.ds(wbase, per_w)])

    return sc_kernel(h, src, dst)


def kernel(h, edge_index):
    src = edge_index[0].astype(jnp.int32)
    dst = edge_index[1].astype(jnp.int32)
    return _dot_predict_sc(h, src, dst, src.shape[0])
